# baseline (speedup 1.0000x reference)
import functools

import jax
import jax.numpy as jnp
from jax import lax
from jax.experimental import pallas as pl
from jax.experimental.pallas import tpu as pltpu
from jax.experimental.pallas import tpu_sc as plsc

_V, _D, _H, _C, _B, _T = 100000, 128, 300, 1000, 1024, 32
_HP = 384
_CP = 1024
_N = _B * _T


def _sc_gather(emb, idx_flat):
    info = plsc.get_sparse_core_info()
    nw = info.num_cores * info.num_subcores
    b_per_w = _N // nw
    ch = 256
    n_ch = b_per_w // ch
    mesh = plsc.VectorSubcoreMesh(core_axis_name="c", subcore_axis_name="s")

    @functools.partial(
        pl.kernel,
        mesh=mesh,
        out_type=jax.ShapeDtypeStruct((_N, _D), jnp.float32),
        scratch_types=[
            pltpu.VMEM((b_per_w,), jnp.int32),
            pltpu.VMEM((2, ch, _D), jnp.float32),
            pltpu.SemaphoreType.DMA,
            pltpu.SemaphoreType.DMA,
            pltpu.SemaphoreType.DMA,
            pltpu.SemaphoreType.DMA,
        ],
    )
    def gather_kernel(table_hbm, idx_hbm, out_hbm, idx_v, rows_v,
                      gs0, gs1, os0, os1):
        wid = lax.axis_index("s") * info.num_cores + lax.axis_index("c")
        base = wid * b_per_w
        pltpu.sync_copy(idx_hbm.at[pl.ds(base, b_per_w)], idx_v)
        gsems, osems = (gs0, gs1), (os0, os1)

        def start_get(c):
            return pltpu.async_copy(
                table_hbm.at[idx_v.at[pl.ds(c * ch, ch)]],
                rows_v.at[c % 2], gsems[c % 2])

        def start_put(c):
            return pltpu.async_copy(
                rows_v.at[c % 2], out_hbm.at[pl.ds(base + c * ch, ch)],
                osems[c % 2])

        gets, puts = [None] * n_ch, [None] * n_ch
        gets[0] = start_get(0)
        if n_ch > 1:
            gets[1] = start_get(1)
        for c in range(n_ch):
            gets[c].wait()
            puts[c] = start_put(c)
            if c + 2 < n_ch:
                puts[c].wait()
                gets[c + 2] = start_get(c + 2)
        for c in range(max(0, n_ch - 2), n_ch):
            puts[c].wait()

    return gather_kernel(emb, idx_flat)


def _tc_gru(x_tm, wrz, win, whn, brz, bn_i, bn_h, wout, bout):
    bblk = 1024
    nb = _B // bblk
    bf = jnp.bfloat16

    def body(x_ref, wrz_ref, win_ref, whn_ref, brz_ref, bni_ref, bnh_ref,
             wout_ref, bout_ref, out_ref):
        w_rz = wrz_ref[...]
        w_in = win_ref[...]
        w_hn = whn_ref[...]
        b_rz = brz_ref[...]
        b_ni = bni_ref[...]
        b_nh = bnh_ref[...]

        half = bblk // 2

        def sub_step(x_t, h):
            h_b = h.astype(bf)
            xh = jnp.concatenate([x_t, h_b], axis=1)
            rz = jnp.dot(xh, w_rz, preferred_element_type=jnp.float32)
            i_n = jnp.dot(x_t, w_in, preferred_element_type=jnp.float32)
            h_n = jnp.dot(h_b, w_hn, preferred_element_type=jnp.float32)
            r = jax.nn.sigmoid(rz[:, :_HP] + b_rz[:, :_HP])
            z = jax.nn.sigmoid(rz[:, _HP:] + b_rz[:, _HP:])
            n = jnp.tanh(i_n + b_ni + r * (h_n + b_nh))
            return (1.0 - z) * n + z * h

        def step(t, hs):
            x_t = x_ref[t].astype(bf)
            return tuple(
                sub_step(x_t[j * half:(j + 1) * half], hs[j])
                for j in range(2))

        h0 = jnp.zeros((half, _HP), jnp.float32)
        hs = lax.fori_loop(0, _T, step, (h0, h0), unroll=2)
        w_o = wout_ref[...]
        b_o = bout_ref[...]
        for j in range(2):
            out_ref[j * half:(j + 1) * half, :] = (
                jnp.dot(hs[j], w_o, preferred_element_type=jnp.float32) + b_o)

    return pl.pallas_call(
        body,
        grid=(nb,),
        in_specs=[
            pl.BlockSpec((_T, bblk, _D), lambda i: (0, i, 0)),
            pl.BlockSpec((_D + _HP, 2 * _HP), lambda i: (0, 0)),
            pl.BlockSpec((_D, _HP), lambda i: (0, 0)),
            pl.BlockSpec((_HP, _HP), lambda i: (0, 0)),
            pl.BlockSpec((1, 2 * _HP), lambda i: (0, 0)),
            pl.BlockSpec((1, _HP), lambda i: (0, 0)),
            pl.BlockSpec((1, _HP), lambda i: (0, 0)),
            pl.BlockSpec((_HP, _CP), lambda i: (0, 0)),
            pl.BlockSpec((1, _CP), lambda i: (0, 0)),
        ],
        out_specs=pl.BlockSpec((bblk, _CP), lambda i: (i, 0)),
        out_shape=jax.ShapeDtypeStruct((_B, _CP), jnp.float32),
    )(x_tm, wrz, win, whn, brz, bn_i, bn_h, wout, bout)


def _pad_gates(w, k):
    parts = jnp.split(w, k, axis=-1)
    pad = [(0, 0)] * (w.ndim - 1) + [(0, _HP - _H)]
    return jnp.concatenate([jnp.pad(p, pad) for p in parts], axis=-1)


def kernel(sequence, emb, W_ih, W_hh, b_ih, b_hh, W_out, b_out):
    idx = jnp.asarray(sequence, jnp.int32).T.reshape(-1)
    x_flat = _sc_gather(emb, idx)
    x_tm = x_flat.reshape(_T, _B, _D)

    bf = jnp.bfloat16
    wrz = jnp.concatenate([
        _pad_gates(W_ih[:, :2 * _H], 2),
        jnp.pad(_pad_gates(W_hh[:, :2 * _H], 2), ((0, _HP - _H), (0, 0))),
    ], axis=0).astype(bf)
    win = _pad_gates(W_ih[:, 2 * _H:], 1).astype(bf)
    whn = jnp.pad(_pad_gates(W_hh[:, 2 * _H:], 1),
                  ((0, _HP - _H), (0, 0))).astype(bf)
    brz = _pad_gates(b_ih[:2 * _H] + b_hh[:2 * _H], 2)[None, :]
    bni = _pad_gates(b_ih[2 * _H:], 1)[None, :]
    bnh = _pad_gates(b_hh[2 * _H:], 1)[None, :]
    wout = jnp.pad(W_out, ((0, _HP - _H), (0, _CP - _C)))
    bout = jnp.pad(b_out, ((0, _CP - _C)))[None, :]

    logits = _tc_gru(x_tm, wrz, win, whn, brz, bni, bnh, wout, bout)
    return logits[:, :_C]


# speedup vs baseline: 1.0057x; 1.0057x over previous
import functools

import jax
import jax.numpy as jnp
from jax import lax
from jax.experimental import pallas as pl
from jax.experimental.pallas import tpu as pltpu
from jax.experimental.pallas import tpu_sc as plsc

_V, _D, _H, _C, _B, _T = 100000, 128, 300, 1000, 1024, 32
_HP = 384
_CP = 1024


def _sc_gather(emb, idx_flat):
    n = idx_flat.shape[0]
    info = plsc.get_sparse_core_info()
    nw = info.num_cores * info.num_subcores
    b_per_w = n // nw
    ch = min(256, b_per_w)
    n_ch = b_per_w // ch
    mesh = plsc.VectorSubcoreMesh(core_axis_name="c", subcore_axis_name="s")

    @functools.partial(
        pl.kernel,
        mesh=mesh,
        out_type=jax.ShapeDtypeStruct((n, _D), jnp.float32),
        scratch_types=[
            pltpu.VMEM((b_per_w,), jnp.int32),
            pltpu.VMEM((2, ch, _D), jnp.float32),
            pltpu.SemaphoreType.DMA,
            pltpu.SemaphoreType.DMA,
            pltpu.SemaphoreType.DMA,
            pltpu.SemaphoreType.DMA,
        ],
    )
    def gather_kernel(table_hbm, idx_hbm, out_hbm, idx_v, rows_v,
                      gs0, gs1, os0, os1):
        wid = lax.axis_index("s") * info.num_cores + lax.axis_index("c")
        base = wid * b_per_w
        pltpu.sync_copy(idx_hbm.at[pl.ds(base, b_per_w)], idx_v)
        gsems, osems = (gs0, gs1), (os0, os1)

        def start_get(c):
            return pltpu.async_copy(
                table_hbm.at[idx_v.at[pl.ds(c * ch, ch)]],
                rows_v.at[c % 2], gsems[c % 2])

        def start_put(c):
            return pltpu.async_copy(
                rows_v.at[c % 2], out_hbm.at[pl.ds(base + c * ch, ch)],
                osems[c % 2])

        gets, puts = [None] * n_ch, [None] * n_ch
        gets[0] = start_get(0)
        if n_ch > 1:
            gets[1] = start_get(1)
        for c in range(n_ch):
            gets[c].wait()
            puts[c] = start_put(c)
            if c + 2 < n_ch:
                puts[c].wait()
                gets[c + 2] = start_get(c + 2)
        for c in range(max(0, n_ch - 2), n_ch):
            puts[c].wait()

    return gather_kernel(emb, idx_flat)


def _tc_gru_chunk(x_tm, h_in, wrz, win, whn, brz, bn_i, bn_h, wout, bout,
                  final):
    t_len = x_tm.shape[0]
    bf = jnp.bfloat16

    def body(x_ref, h0_ref, wrz_ref, win_ref, whn_ref, brz_ref, bni_ref,
             bnh_ref, wout_ref, bout_ref, out_ref):
        w_rz = wrz_ref[...]
        w_in = win_ref[...]
        w_hn = whn_ref[...]
        b_rz = brz_ref[...]
        b_ni = bni_ref[...]
        b_nh = bnh_ref[...]

        def step(t, h):
            x_t = x_ref[t].astype(bf)
            h_b = h.astype(bf)
            xh = jnp.concatenate([x_t, h_b], axis=1)
            rz = jnp.dot(xh, w_rz, preferred_element_type=jnp.float32)
            i_n = jnp.dot(x_t, w_in, preferred_element_type=jnp.float32)
            h_n = jnp.dot(h_b, w_hn, preferred_element_type=jnp.float32)
            r = jax.nn.sigmoid(rz[:, :_HP] + b_rz[:, :_HP])
            z = jax.nn.sigmoid(rz[:, _HP:] + b_rz[:, _HP:])
            n = jnp.tanh(i_n + b_ni + r * (h_n + b_nh))
            return (1.0 - z) * n + z * h

        h = lax.fori_loop(0, t_len, step, h0_ref[...], unroll=2)
        if final:
            out_ref[...] = (
                jnp.dot(h, wout_ref[...], preferred_element_type=jnp.float32)
                + bout_ref[...])
        else:
            out_ref[...] = h

    out_shape = (_B, _CP) if final else (_B, _HP)
    full = lambda s: pl.BlockSpec(s, lambda: tuple(0 for _ in s))
    return pl.pallas_call(
        body,
        in_specs=[
            full((t_len, _B, _D)),
            full((_B, _HP)),
            full((_D + _HP, 2 * _HP)),
            full((_D, _HP)),
            full((_HP, _HP)),
            full((1, 2 * _HP)),
            full((1, _HP)),
            full((1, _HP)),
            full((_HP, _CP)),
            full((1, _CP)),
        ],
        out_specs=full(out_shape),
        out_shape=jax.ShapeDtypeStruct(out_shape, jnp.float32),
    )(x_tm, h_in, wrz, win, whn, brz, bn_i, bn_h, wout, bout)


def _pad_gates(w, k):
    parts = jnp.split(w, k, axis=-1)
    pad = [(0, 0)] * (w.ndim - 1) + [(0, _HP - _H)]
    return jnp.concatenate([jnp.pad(p, pad) for p in parts], axis=-1)


def kernel(sequence, emb, W_ih, W_hh, b_ih, b_hh, W_out, b_out):
    idx = jnp.asarray(sequence, jnp.int32).T.reshape(-1)

    bf = jnp.bfloat16
    wrz = jnp.concatenate([
        _pad_gates(W_ih[:, :2 * _H], 2),
        jnp.pad(_pad_gates(W_hh[:, :2 * _H], 2), ((0, _HP - _H), (0, 0))),
    ], axis=0).astype(bf)
    win = _pad_gates(W_ih[:, 2 * _H:], 1).astype(bf)
    whn = jnp.pad(_pad_gates(W_hh[:, 2 * _H:], 1),
                  ((0, _HP - _H), (0, 0))).astype(bf)
    brz = _pad_gates(b_ih[:2 * _H] + b_hh[:2 * _H], 2)[None, :]
    bni = _pad_gates(b_ih[2 * _H:], 1)[None, :]
    bnh = _pad_gates(b_hh[2 * _H:], 1)[None, :]
    wout = jnp.pad(W_out, ((0, _HP - _H), (0, _CP - _C)))
    bout = jnp.pad(b_out, ((0, _CP - _C)))[None, :]

    t_half = _T // 2
    n_half = t_half * _B
    x0 = _sc_gather(emb, idx[:n_half]).reshape(t_half, _B, _D)
    x1 = _sc_gather(emb, idx[n_half:]).reshape(t_half, _B, _D)

    h = jnp.zeros((_B, _HP), jnp.float32)
    h = _tc_gru_chunk(x0, h, wrz, win, whn, brz, bni, bnh, wout, bout,
                      final=False)
    logits = _tc_gru_chunk(x1, h, wrz, win, whn, brz, bni, bnh, wout, bout,
                           final=True)
    return logits[:, :_C]


# speedup vs baseline: 1.0338x; 1.0279x over previous
import functools

import jax
import jax.numpy as jnp
from jax import lax
from jax.experimental import pallas as pl
from jax.experimental.pallas import tpu as pltpu
from jax.experimental.pallas import tpu_sc as plsc

_V, _D, _H, _C, _B, _T = 100000, 128, 300, 1000, 1024, 32
_HP = 384
_CP = 1024


def _sc_gather(emb, idx_flat):
    n = idx_flat.shape[0]
    info = plsc.get_sparse_core_info()
    nw = info.num_cores * info.num_subcores
    b_per_w = n // nw
    ch = min(256, b_per_w)
    n_ch = b_per_w // ch
    mesh = plsc.VectorSubcoreMesh(core_axis_name="c", subcore_axis_name="s")

    @functools.partial(
        pl.kernel,
        mesh=mesh,
        out_type=jax.ShapeDtypeStruct((n, _D), jnp.float32),
        scratch_types=[
            pltpu.VMEM((b_per_w,), jnp.int32),
            pltpu.VMEM((2, ch, _D), jnp.float32),
            pltpu.SemaphoreType.DMA,
            pltpu.SemaphoreType.DMA,
            pltpu.SemaphoreType.DMA,
            pltpu.SemaphoreType.DMA,
        ],
    )
    def gather_kernel(table_hbm, idx_hbm, out_hbm, idx_v, rows_v,
                      gs0, gs1, os0, os1):
        wid = lax.axis_index("s") * info.num_cores + lax.axis_index("c")
        base = wid * b_per_w
        pltpu.sync_copy(idx_hbm.at[pl.ds(base, b_per_w)], idx_v)
        gsems, osems = (gs0, gs1), (os0, os1)

        def start_get(c):
            return pltpu.async_copy(
                table_hbm.at[idx_v.at[pl.ds(c * ch, ch)]],
                rows_v.at[c % 2], gsems[c % 2])

        def start_put(c):
            return pltpu.async_copy(
                rows_v.at[c % 2], out_hbm.at[pl.ds(base + c * ch, ch)],
                osems[c % 2])

        gets, puts = [None] * n_ch, [None] * n_ch
        gets[0] = start_get(0)
        if n_ch > 1:
            gets[1] = start_get(1)
        for c in range(n_ch):
            gets[c].wait()
            puts[c] = start_put(c)
            if c + 2 < n_ch:
                puts[c].wait()
                gets[c + 2] = start_get(c + 2)
        for c in range(max(0, n_ch - 2), n_ch):
            puts[c].wait()

    return gather_kernel(emb, idx_flat)


def _tc_gru_chunk(x_tm, h_in, wrz, win, whn, bn_i, wout, bout, final):
    t_len = x_tm.shape[0]
    bf = jnp.bfloat16

    def body(x_ref, h0_ref, wrz_ref, win_ref, whn_ref, bni_ref,
             wout_ref, bout_ref, out_ref):
        w_rz = wrz_ref[...]
        w_in = win_ref[...]
        w_hn = whn_ref[...]
        b_ni = bni_ref[...]

        def step(t, h):
            x_t = x_ref[t].astype(bf)
            h_b = h.astype(bf)
            xh = jnp.concatenate([x_t, h_b], axis=1)
            rz = jnp.dot(xh, w_rz, preferred_element_type=jnp.float32)
            i_n = jnp.dot(x_t, w_in, preferred_element_type=jnp.float32)
            h_n = jnp.dot(h_b, w_hn, preferred_element_type=jnp.float32)
            r = jax.nn.sigmoid(rz[:, :_HP])
            z = jax.nn.sigmoid(rz[:, _HP:])
            n = jnp.tanh(i_n + b_ni + r * h_n)
            return n + z * (h - n)

        h = lax.fori_loop(0, t_len, step, h0_ref[...], unroll=2)
        if final:
            out_ref[...] = (
                jnp.dot(h, wout_ref[...], preferred_element_type=jnp.float32)
                + bout_ref[...])
        else:
            out_ref[...] = h

    out_shape = (_B, _CP) if final else (_B, _HP)
    return pl.pallas_call(
        body,
        out_shape=jax.ShapeDtypeStruct(out_shape, jnp.float32),
    )(x_tm, h_in, wrz, win, whn, bn_i, wout, bout)


def _pad_gates(w, k):
    parts = jnp.split(w, k, axis=-1)
    pad = [(0, 0)] * (w.ndim - 1) + [(0, _HP - _H)]
    return jnp.concatenate([jnp.pad(p, pad) for p in parts], axis=-1)


def kernel(sequence, emb, W_ih, W_hh, b_ih, b_hh, W_out, b_out):
    idx = jnp.asarray(sequence, jnp.int32).T.reshape(-1)

    bf = jnp.bfloat16
    brz_row = _pad_gates(b_ih[:2 * _H] + b_hh[:2 * _H], 2)
    brz_row = brz_row.at[_HP + _H].set(20.0)
    wrz = jnp.concatenate([
        _pad_gates(W_ih[:, :2 * _H], 2),
        jnp.pad(_pad_gates(W_hh[:, :2 * _H], 2), ((0, _HP - _H), (0, 0)))
        .at[_H].set(brz_row),
    ], axis=0).astype(bf)
    win = _pad_gates(W_ih[:, 2 * _H:], 1).astype(bf)
    whn = (jnp.pad(_pad_gates(W_hh[:, 2 * _H:], 1),
                   ((0, _HP - _H), (0, 0)))
           .at[_H].set(_pad_gates(b_hh[2 * _H:], 1))).astype(bf)
    bni = _pad_gates(b_ih[2 * _H:], 1)[None, :]
    wout = jnp.pad(W_out, ((0, _HP - _H), (0, _CP - _C)))
    bout = jnp.pad(b_out, ((0, _CP - _C)))[None, :]

    x = _sc_gather(emb, idx).reshape(_T, _B, _D)
    h0 = jnp.zeros((_B, _HP), jnp.float32).at[:, _H].set(1.0)
    logits = _tc_gru_chunk(x, h0, wrz, win, whn, bni, wout, bout, final=True)
    return logits[:, :_C]
